# baseline (speedup 1.0000x reference)
import functools

import jax
import jax.numpy as jnp
from jax import lax
from jax.experimental import pallas as pl

BATCH = 16
NUM_ROIS = 2000
NUM_CLASSES = 81


CB = 27
N_CI = NUM_CLASSES // CB


def _loss_body(yt_ref, cid_ref, box_ref, o_ref):
    b = pl.program_id(0)
    ci = pl.program_id(1)

    @pl.when(jnp.logical_and(b == 0, ci == 0))
    def _():
        o_ref[...] = jnp.zeros_like(o_ref)

    blk = yt_ref[0]
    cid4 = cid_ref[0]
    box = box_ref[0]

    cls_ids = ci * CB + lax.broadcasted_iota(jnp.int32, (CB, 1, 1), 0)
    m = cid4[None] == cls_ids
    sel = jnp.sum(jnp.where(m, blk, 0.0), axis=0)
    lo = jnp.maximum(ci * CB, 1)
    inblk = jnp.logical_and(cid4 >= lo, cid4 < (ci + 1) * CB)
    mrow = jnp.where(inblk, 1.0, 0.0)
    diff = jnp.abs(box - sel) * mrow
    dlo = jnp.minimum(diff, 1.0)
    loss = 0.5 * dlo * dlo + jnp.maximum(diff, 1.0) - 1.0
    s = jnp.sum(loss)
    cnt = jnp.sum(mrow)

    lane = lax.broadcasted_iota(jnp.int32, (1, 128), 1)
    upd = jnp.where(lane == 0, s, jnp.where(lane == 1, cnt, 0.0))
    o_ref[...] += upd


@jax.jit
def kernel(target_bounding_box, target_class_ids, y_pred):
    yt = jnp.transpose(y_pred, (0, 2, 3, 1))
    bt = jnp.transpose(target_bounding_box, (0, 2, 1))
    cid = target_class_ids.astype(jnp.int32)
    cid4 = jnp.broadcast_to(cid[:, None, :], (BATCH, 4, NUM_ROIS))

    out = pl.pallas_call(
        _loss_body,
        grid=(BATCH, N_CI),
        in_specs=[
            pl.BlockSpec((1, CB, 4, NUM_ROIS),
                         lambda b, ci: (b, ci, 0, 0)),
            pl.BlockSpec((1, 4, NUM_ROIS), lambda b, ci: (b, 0, 0)),
            pl.BlockSpec((1, 4, NUM_ROIS), lambda b, ci: (b, 0, 0)),
        ],
        out_specs=pl.BlockSpec((1, 128), lambda b, ci: (0, 0)),
        out_shape=jax.ShapeDtypeStruct((1, 128), jnp.float32),
    )(yt, cid4, bt)

    total = out[0, 0]
    cnt = out[0, 1]
    return jnp.where(cnt > 0, total / cnt, jnp.float32(0.0))


# speedup vs baseline: 1.7948x; 1.7948x over previous
import functools

import jax
import jax.numpy as jnp
from jax import lax
from jax.experimental import pallas as pl

BATCH = 16
NUM_ROIS = 2000
NUM_CLASSES = 81
BB = 2


def _loss_body(yt_ref, cid_ref, box_ref, o_ref):
    b = pl.program_id(0)

    @pl.when(b == 0)
    def _():
        o_ref[...] = jnp.zeros_like(o_ref)

    blk = yt_ref[...]
    cid4 = cid_ref[...]
    box = box_ref[...]

    cls_ids = lax.broadcasted_iota(jnp.int32, (1, NUM_CLASSES, 1, 1), 1)
    m = cid4[:, None] == cls_ids
    pred = jnp.sum(jnp.where(m, blk, 0.0), axis=1)

    mrow = jnp.minimum(cid4, 1).astype(jnp.float32)
    diff = jnp.abs(box - pred) * mrow
    dlo = jnp.minimum(diff, 1.0)
    loss = 0.5 * dlo * dlo + jnp.maximum(diff, 1.0) - 1.0
    s = jnp.sum(loss)
    cnt = jnp.sum(mrow)

    lane = lax.broadcasted_iota(jnp.int32, (1, 128), 1)
    upd = jnp.where(lane == 0, s, jnp.where(lane == 1, cnt, 0.0))
    o_ref[...] += upd


@jax.jit
def kernel(target_bounding_box, target_class_ids, y_pred):
    yt = jnp.transpose(y_pred, (0, 2, 3, 1))
    bt = jnp.transpose(target_bounding_box, (0, 2, 1))
    cid = target_class_ids.astype(jnp.int32)
    cid4 = jnp.broadcast_to(cid[:, None, :], (BATCH, 4, NUM_ROIS))

    out = pl.pallas_call(
        _loss_body,
        grid=(BATCH // BB,),
        in_specs=[
            pl.BlockSpec((BB, NUM_CLASSES, 4, NUM_ROIS),
                         lambda b: (b, 0, 0, 0)),
            pl.BlockSpec((BB, 4, NUM_ROIS), lambda b: (b, 0, 0)),
            pl.BlockSpec((BB, 4, NUM_ROIS), lambda b: (b, 0, 0)),
        ],
        out_specs=pl.BlockSpec((1, 128), lambda b: (0, 0)),
        out_shape=jax.ShapeDtypeStruct((1, 128), jnp.float32),
    )(yt, cid4, bt)

    total = out[0, 0]
    cnt = out[0, 1]
    return jnp.where(cnt > 0, total / cnt, jnp.float32(0.0))


# speedup vs baseline: 1.9488x; 1.0858x over previous
import functools

import jax
import jax.numpy as jnp
from jax import lax
from jax.experimental import pallas as pl

BATCH = 16
NUM_ROIS = 2000
NUM_CLASSES = 81
BB = 4


def _loss_body(yt_ref, cid_ref, box_ref, o_ref):
    b = pl.program_id(0)

    @pl.when(b == 0)
    def _():
        o_ref[...] = jnp.zeros_like(o_ref)

    blk = yt_ref[...]
    cid4 = cid_ref[...]
    box = box_ref[...]

    cls_ids = lax.broadcasted_iota(jnp.int32, (1, NUM_CLASSES, 1, 1), 1)
    m = cid4[:, None] == cls_ids
    pred = jnp.sum(jnp.where(m, blk, 0.0), axis=1)

    mrow = jnp.minimum(cid4, 1).astype(jnp.float32)
    diff = jnp.abs(box - pred) * mrow
    dlo = jnp.minimum(diff, 1.0)
    loss = 0.5 * dlo * dlo + jnp.maximum(diff, 1.0) - 1.0
    s = jnp.sum(loss)
    cnt = jnp.sum(mrow)

    lane = lax.broadcasted_iota(jnp.int32, (1, 128), 1)
    upd = jnp.where(lane == 0, s, jnp.where(lane == 1, cnt, 0.0))
    o_ref[...] += upd


@jax.jit
def kernel(target_bounding_box, target_class_ids, y_pred):
    yt = jnp.transpose(y_pred, (0, 2, 3, 1))
    bt = jnp.transpose(target_bounding_box, (0, 2, 1))
    cid = target_class_ids.astype(jnp.int32)
    cid4 = jnp.broadcast_to(cid[:, None, :], (BATCH, 4, NUM_ROIS))

    out = pl.pallas_call(
        _loss_body,
        grid=(BATCH // BB,),
        in_specs=[
            pl.BlockSpec((BB, NUM_CLASSES, 4, NUM_ROIS),
                         lambda b: (b, 0, 0, 0)),
            pl.BlockSpec((BB, 4, NUM_ROIS), lambda b: (b, 0, 0)),
            pl.BlockSpec((BB, 4, NUM_ROIS), lambda b: (b, 0, 0)),
        ],
        out_specs=pl.BlockSpec((1, 128), lambda b: (0, 0)),
        out_shape=jax.ShapeDtypeStruct((1, 128), jnp.float32),
    )(yt, cid4, bt)

    total = out[0, 0]
    cnt = out[0, 1]
    return jnp.where(cnt > 0, total / cnt, jnp.float32(0.0))


# speedup vs baseline: 1.9628x; 1.0072x over previous
import functools

import jax
import jax.numpy as jnp
from jax import lax
from jax.experimental import pallas as pl

BATCH = 16
NUM_ROIS = 2000
NUM_CLASSES = 81
BB = 4


def _loss_body(yt_ref, cid_ref, box_ref, o_ref):
    b = pl.program_id(0)

    @pl.when(b == 0)
    def _():
        o_ref[...] = jnp.zeros_like(o_ref)

    blk = yt_ref[...]
    cid4 = cid_ref[...].astype(jnp.int32)
    box = box_ref[...]

    cls_ids = lax.broadcasted_iota(jnp.int32, (1, NUM_CLASSES, 1, 1), 1)
    m = cid4[:, None] == cls_ids
    pred = jnp.sum(jnp.where(m, blk, 0.0), axis=1)

    mrow = jnp.minimum(cid4, 1).astype(jnp.float32)
    diff = jnp.abs(box - pred) * mrow
    dlo = jnp.minimum(diff, 1.0)
    loss = 0.5 * dlo * dlo + jnp.maximum(diff, 1.0) - 1.0
    s = jnp.sum(loss)
    cnt = jnp.sum(mrow)

    lane = lax.broadcasted_iota(jnp.int32, (1, 128), 1)
    upd = jnp.where(lane == 0, s, jnp.where(lane == 1, cnt, 0.0))
    o_ref[...] += upd


@jax.jit
def kernel(target_bounding_box, target_class_ids, y_pred):
    yt = jnp.transpose(y_pred, (0, 2, 3, 1))
    bt = jnp.transpose(target_bounding_box, (0, 2, 1))
    cid = target_class_ids.astype(jnp.int8)
    cid4 = jnp.broadcast_to(cid[:, None, :], (BATCH, 4, NUM_ROIS))

    out = pl.pallas_call(
        _loss_body,
        grid=(BATCH // BB,),
        in_specs=[
            pl.BlockSpec((BB, NUM_CLASSES, 4, NUM_ROIS),
                         lambda b: (b, 0, 0, 0)),
            pl.BlockSpec((BB, 4, NUM_ROIS), lambda b: (b, 0, 0)),
            pl.BlockSpec((BB, 4, NUM_ROIS), lambda b: (b, 0, 0)),
        ],
        out_specs=pl.BlockSpec((1, 128), lambda b: (0, 0)),
        out_shape=jax.ShapeDtypeStruct((1, 128), jnp.float32),
    )(yt, cid4, bt)

    total = out[0, 0]
    cnt = out[0, 1]
    return jnp.where(cnt > 0, total / cnt, jnp.float32(0.0))


# speedup vs baseline: 1.9673x; 1.0023x over previous
import jax
import jax.numpy as jnp
from jax import lax
from jax.experimental import pallas as pl

BATCH = 16
NUM_ROIS = 2000
NUM_CLASSES = 81
BB = 4


def _loss_body(yt_ref, cid_ref, box_ref, o_ref):
    b = pl.program_id(0)

    @pl.when(b == 0)
    def _():
        o_ref[...] = jnp.zeros_like(o_ref)

    blk = yt_ref[...]
    cid4 = cid_ref[...].astype(jnp.int32)
    box = box_ref[...]

    cls_ids = lax.broadcasted_iota(jnp.int32, (1, NUM_CLASSES, 1, 1), 1)
    m = cid4[:, None] == cls_ids
    pred = jnp.sum(jnp.where(m, blk, 0.0), axis=1)

    mrow = jnp.minimum(cid4, 1).astype(jnp.float32)
    diff = jnp.abs(box - pred) * mrow
    dlo = jnp.minimum(diff, 1.0)
    loss = 0.5 * dlo * dlo + jnp.maximum(diff, 1.0) - 1.0
    s = jnp.sum(loss)
    cnt = jnp.sum(mrow)

    lane = lax.broadcasted_iota(jnp.int32, (1, 128), 1)
    upd = jnp.where(lane == 0, s, jnp.where(lane == 1, cnt, 0.0))
    o_ref[...] += upd


@jax.jit
def kernel(target_bounding_box, target_class_ids, y_pred):
    yt = jnp.transpose(y_pred, (0, 2, 3, 1))
    bt = jnp.transpose(target_bounding_box, (0, 2, 1))
    cid = target_class_ids.astype(jnp.int8)
    cid4 = jnp.broadcast_to(cid[:, None, :], (BATCH, 4, NUM_ROIS))

    out = pl.pallas_call(
        _loss_body,
        grid=(BATCH // BB,),
        in_specs=[
            pl.BlockSpec((BB, NUM_CLASSES, 4, NUM_ROIS),
                         lambda b: (b, 0, 0, 0)),
            pl.BlockSpec((BB, 4, NUM_ROIS), lambda b: (b, 0, 0)),
            pl.BlockSpec((BB, 4, NUM_ROIS), lambda b: (b, 0, 0)),
        ],
        out_specs=pl.BlockSpec((1, 128), lambda b: (0, 0)),
        out_shape=jax.ShapeDtypeStruct((1, 128), jnp.float32),
    )(yt, cid4, bt)

    total = out[0, 0]
    cnt = out[0, 1]
    return jnp.where(cnt > 0, total / cnt, jnp.float32(0.0))
